# baseline (speedup 1.0000x reference)
import functools

import jax
import jax.numpy as jnp
from jax.experimental import pallas as pl
from jax.experimental.pallas import tpu as pltpu

_BM = 256


def _body(x_ref, W1_ref, b1_ref, W2_ref, b2_ref, W3_ref, b3_ref,
          Ws_ref, bs_ref, Wc1a_ref, Wc1b_ref, bc1_ref, Wc2_ref, bc2_ref,
          wc3_ref, bc3_ref, out_ref):
    x = x_ref[...]
    T = x.shape[1]

    mean = jnp.mean(x, axis=1, keepdims=True)
    centered = x - mean
    c2 = centered * centered
    var_unb = jnp.sum(c2, axis=1, keepdims=True) / (T - 1)
    std = jnp.sqrt(var_unb)
    m3 = jnp.mean(c2 * centered, axis=1, keepdims=True)
    m4 = jnp.mean(c2 * c2, axis=1, keepdims=True)
    skew = m3 / (std * std * std + 1e-8)
    kurt = m4 / (std * std * std * std + 1e-8)
    xmin = jnp.min(x, axis=1, keepdims=True)
    xmax = jnp.max(x, axis=1, keepdims=True)

    stats = (mean, std, xmin, xmax, skew, kurt)
    statf = bs_ref[...]
    for i, s in enumerate(stats):
        statf = statf + s * Ws_ref[i:i + 1, :]

    xb = x.astype(jnp.bfloat16)
    h = jnp.dot(xb, W1_ref[...], preferred_element_type=jnp.float32)
    h = jax.nn.relu(h + b1_ref[...]).astype(jnp.bfloat16)
    h = jnp.dot(h, W2_ref[...], preferred_element_type=jnp.float32)
    h = jax.nn.relu(h + b2_ref[...]).astype(jnp.bfloat16)
    h = jnp.dot(h, W3_ref[...], preferred_element_type=jnp.float32)
    seq = jax.nn.relu(h + b3_ref[...]).astype(jnp.bfloat16)

    c = (jnp.dot(seq, Wc1a_ref[...], preferred_element_type=jnp.float32)
         + jnp.dot(statf.astype(jnp.bfloat16), Wc1b_ref[...],
                   preferred_element_type=jnp.float32))
    c = jax.nn.relu(c + bc1_ref[...]).astype(jnp.bfloat16)
    c = jnp.dot(c, Wc2_ref[...], preferred_element_type=jnp.float32)
    c = jax.nn.relu(c + bc2_ref[...])

    z = jnp.sum(c * wc3_ref[...], axis=1, keepdims=True) + bc3_ref[...]
    out_ref[...] = jax.nn.sigmoid(z) * 4.0 + 6.0


@jax.jit
def kernel(x, W1, b1, W2, b2, W3, b3, Ws, bs, Wc1, bc1, Wc2, bc2, Wc3, bc3):
    B, T = x.shape
    nb = B // _BM

    W1b = W1.astype(jnp.bfloat16)
    W2b = W2.astype(jnp.bfloat16)
    W3b = W3.astype(jnp.bfloat16)
    Wc1a = Wc1[:128].astype(jnp.bfloat16)
    Wc1b = Wc1[128:].astype(jnp.bfloat16)
    Wc2b = Wc2.astype(jnp.bfloat16)
    wc3 = Wc3.reshape(1, -1)
    b1r = b1.reshape(1, -1)
    b2r = b2.reshape(1, -1)
    b3r = b3.reshape(1, -1)
    bsr = bs.reshape(1, -1)
    bc1r = bc1.reshape(1, -1)
    bc2r = bc2.reshape(1, -1)
    bc3r = bc3.reshape(1, -1)

    full = lambda a: pl.BlockSpec(a.shape, lambda i: (0,) * a.ndim)
    out = pl.pallas_call(
        _body,
        grid=(nb,),
        in_specs=[
            pl.BlockSpec((_BM, T), lambda i: (i, 0)),
            full(W1b), full(b1r), full(W2b), full(b2r), full(W3b), full(b3r),
            full(Ws), full(bsr), full(Wc1a), full(Wc1b), full(bc1r),
            full(Wc2b), full(bc2r), full(wc3), full(bc3r),
        ],
        out_specs=pl.BlockSpec((_BM, 1), lambda i: (i, 0)),
        out_shape=jax.ShapeDtypeStruct((B, 1), jnp.float32),
        compiler_params=pltpu.CompilerParams(
            dimension_semantics=("parallel",),
        ),
    )(x, W1b, b1r, W2b, b2r, W3b, b3r, Ws, bsr, Wc1a, Wc1b, bc1r,
      Wc2b, bc2r, wc3, bc3r)
    return out.reshape(B)


# speedup vs baseline: 1.4891x; 1.4891x over previous
import functools

import jax
import jax.numpy as jnp
from jax.experimental import pallas as pl
from jax.experimental.pallas import tpu as pltpu

_BM = 1024


def _body(x_ref, W1_ref, b1_ref, W2_ref, b2_ref, W3_ref, b3_ref,
          Ws_ref, bs_ref, Wc1a_ref, Wc1b_ref, bc1_ref, Wc2_ref, bc2_ref,
          wc3_ref, bc3_ref, out_ref):
    x = x_ref[...]
    T = x.shape[1]

    mean = jnp.mean(x, axis=1, keepdims=True)
    centered = x - mean
    c2 = centered * centered
    var_unb = jnp.sum(c2, axis=1, keepdims=True) / (T - 1)
    std = jnp.sqrt(var_unb)
    m3 = jnp.mean(c2 * centered, axis=1, keepdims=True)
    m4 = jnp.mean(c2 * c2, axis=1, keepdims=True)
    skew = m3 / (std * std * std + 1e-8)
    kurt = m4 / (std * std * std * std + 1e-8)
    xmin = jnp.min(x, axis=1, keepdims=True)
    xmax = jnp.max(x, axis=1, keepdims=True)

    stats = (mean, std, xmin, xmax, skew, kurt)
    statf = bs_ref[...]
    for i, s in enumerate(stats):
        statf = statf + s * Ws_ref[i:i + 1, :]

    xb = x.astype(jnp.bfloat16)
    h = jnp.dot(xb, W1_ref[...], preferred_element_type=jnp.float32)
    h = jax.nn.relu(h + b1_ref[...]).astype(jnp.bfloat16)
    h = jnp.dot(h, W2_ref[...], preferred_element_type=jnp.float32)
    h = jax.nn.relu(h + b2_ref[...]).astype(jnp.bfloat16)
    h = jnp.dot(h, W3_ref[...], preferred_element_type=jnp.float32)
    seq = jax.nn.relu(h + b3_ref[...]).astype(jnp.bfloat16)

    c = (jnp.dot(seq, Wc1a_ref[...], preferred_element_type=jnp.float32)
         + jnp.dot(statf.astype(jnp.bfloat16), Wc1b_ref[...],
                   preferred_element_type=jnp.float32))
    c = jax.nn.relu(c + bc1_ref[...]).astype(jnp.bfloat16)
    c = jnp.dot(c, Wc2_ref[...], preferred_element_type=jnp.float32)
    c = jax.nn.relu(c + bc2_ref[...])

    z = jnp.sum(c * wc3_ref[...], axis=1, keepdims=True) + bc3_ref[...]
    out_ref[...] = jax.nn.sigmoid(z) * 4.0 + 6.0


@jax.jit
def kernel(x, W1, b1, W2, b2, W3, b3, Ws, bs, Wc1, bc1, Wc2, bc2, Wc3, bc3):
    B, T = x.shape
    nb = B // _BM

    W1b = W1.astype(jnp.bfloat16)
    W2b = W2.astype(jnp.bfloat16)
    W3b = W3.astype(jnp.bfloat16)
    Wc1a = Wc1[:128].astype(jnp.bfloat16)
    Wc1b = Wc1[128:].astype(jnp.bfloat16)
    Wc2b = Wc2.astype(jnp.bfloat16)
    wc3 = Wc3.reshape(1, -1)
    b1r = b1.reshape(1, -1)
    b2r = b2.reshape(1, -1)
    b3r = b3.reshape(1, -1)
    bsr = bs.reshape(1, -1)
    bc1r = bc1.reshape(1, -1)
    bc2r = bc2.reshape(1, -1)
    bc3r = bc3.reshape(1, -1)

    full = lambda a: pl.BlockSpec(a.shape, lambda i: (0,) * a.ndim)
    out = pl.pallas_call(
        _body,
        grid=(nb,),
        in_specs=[
            pl.BlockSpec((_BM, T), lambda i: (i, 0)),
            full(W1b), full(b1r), full(W2b), full(b2r), full(W3b), full(b3r),
            full(Ws), full(bsr), full(Wc1a), full(Wc1b), full(bc1r),
            full(Wc2b), full(bc2r), full(wc3), full(bc3r),
        ],
        out_specs=pl.BlockSpec((_BM, 1), lambda i: (i, 0)),
        out_shape=jax.ShapeDtypeStruct((B, 1), jnp.float32),
        compiler_params=pltpu.CompilerParams(
            dimension_semantics=("parallel",),
        ),
    )(x, W1b, b1r, W2b, b2r, W3b, b3r, Ws, bsr, Wc1a, Wc1b, bc1r,
      Wc2b, bc2r, wc3, bc3r)
    return out.reshape(B)
